# Initial kernel scaffold; baseline (speedup 1.0000x reference)
import jax
import jax.numpy as jnp
from jax.experimental import pallas as pl


def kernel(actor_emb, topic_ids, Wa, ba, table, Wt, bt, scale):
    raise NotImplementedError("write your pallas kernel here")


# speedup vs baseline: 2.2656x; 2.2656x over previous
import functools

import jax
import jax.numpy as jnp
from jax import lax
from jax.experimental import pallas as pl
from jax.experimental.pallas import tpu as pltpu
from jax.experimental.pallas import tpu_sc as plsc

D = 128
NC = 2
NS = 16
NW = NC * NS
CH = 128


def _sc_gather(table, ids3):
  n_ch = ids3.shape[1]
  b_per_w = n_ch * CH
  total = NW * b_per_w
  mesh = plsc.VectorSubcoreMesh(
      core_axis_name="c", subcore_axis_name="s", num_cores=NC, num_subcores=NS
  )

  @functools.partial(
      pl.kernel,
      out_type=jax.ShapeDtypeStruct((total, D), jnp.float32),
      mesh=mesh,
      scratch_types=[
          pltpu.VMEM((n_ch, CH), jnp.int32),
          pltpu.VMEM((CH, D), jnp.float32),
          pltpu.VMEM((CH, D), jnp.float32),
          pltpu.SemaphoreType.DMA,
          pltpu.SemaphoreType.DMA,
      ],
  )
  def k(table_hbm, ids_hbm, out_hbm, idx_v, rows0, rows1, sem0, sem1):
    wid = lax.axis_index("s") * NC + lax.axis_index("c")
    base = wid * b_per_w
    pltpu.sync_copy(ids_hbm.at[wid], idx_v)
    pltpu.async_copy(table_hbm.at[idx_v.at[0]], rows0, sem0)
    pltpu.async_copy(table_hbm.at[idx_v.at[1]], rows1, sem1)

    def pair(p, _):
      g0 = 2 * p

      pltpu.make_async_copy(table_hbm.at[idx_v.at[g0]], rows0, sem0).wait()
      pltpu.sync_copy(rows0, out_hbm.at[pl.ds(base + g0 * CH, CH)])

      @pl.when(g0 + 2 < n_ch)
      def _():
        pltpu.async_copy(table_hbm.at[idx_v.at[g0 + 2]], rows0, sem0)

      pltpu.make_async_copy(table_hbm.at[idx_v.at[g0 + 1]], rows1, sem1).wait()
      pltpu.sync_copy(rows1, out_hbm.at[pl.ds(base + (g0 + 1) * CH, CH)])

      @pl.when(g0 + 3 < n_ch)
      def _():
        pltpu.async_copy(table_hbm.at[idx_v.at[g0 + 3]], rows1, sem1)

      return ()

    lax.fori_loop(0, n_ch // 2, pair, ())

  return k(table, ids3)


def _tc_main(x, g, wa, ba, wt, bt, rows_per_tile):
  bl = x.shape[0]
  nt = bl // rows_per_tile
  r = rows_per_tile

  def body(x_ref, g_ref, wa_ref, ba_ref, wt_ref, bt_ref, out_ref, m_s):
    @pl.when(pl.program_id(0) == 0)
    def _():
      m_s[...] = lax.dot_general(
          wa_ref[...], wt_ref[...], (((0,), (0,)), ((), ())),
          preferred_element_type=jnp.float32)

    xv = x_ref[...]
    gv = g_ref[...]
    v = jnp.dot(ba_ref[...], wt_ref[...], preferred_element_type=jnp.float32)
    u = jnp.dot(bt_ref[...], wa_ref[...], preferred_element_type=jnp.float32)
    c = jnp.sum(ba_ref[...] * bt_ref[...])
    a = jnp.dot(xv, m_s[...], preferred_element_type=jnp.float32) + v
    res = jnp.sum(a * gv, axis=1) + jnp.sum(xv * u, axis=1) + c
    out_ref[...] = res.reshape(1, 1, r)

  out = pl.pallas_call(
      body,
      grid=(nt,),
      in_specs=[
          pl.BlockSpec((r, D), lambda i: (i, 0)),
          pl.BlockSpec((r, D), lambda i: (i, 0)),
          pl.BlockSpec((D, D), lambda i: (0, 0)),
          pl.BlockSpec((1, D), lambda i: (0, 0)),
          pl.BlockSpec((D, D), lambda i: (0, 0)),
          pl.BlockSpec((1, D), lambda i: (0, 0)),
      ],
      out_specs=pl.BlockSpec((1, 1, r), lambda i: (i, 0, 0)),
      out_shape=jax.ShapeDtypeStruct((nt, 1, r), jnp.float32),
      scratch_shapes=[pltpu.VMEM((D, D), jnp.float32)],
  )(x, g, wa, ba, wt, bt)
  return out.reshape(bl)


def kernel(actor_emb, topic_ids, Wa, ba, table, Wt, bt, scale):
  b, l, d = actor_emb.shape
  bl = b * l
  n_ch = bl // (NW * CH)

  wa_s = Wa * scale
  ba_s = (ba * scale).reshape(1, d)

  ids3 = topic_ids.reshape(NW, n_ch, CH).astype(jnp.int32)
  gathered = _sc_gather(table, ids3)
  x = actor_emb.reshape(bl, d)
  out = _tc_main(x, gathered, wa_s, ba_s, Wt, bt.reshape(1, d), 2048)
  return out.reshape(b, l)


# speedup vs baseline: 2.2762x; 1.0047x over previous
import functools

import jax
import jax.numpy as jnp
from jax import lax
from jax.experimental import pallas as pl
from jax.experimental.pallas import tpu as pltpu
from jax.experimental.pallas import tpu_sc as plsc

D = 128
NC = 2
NS = 16
NW = NC * NS
CH = 128


def _sc_gather(table, ids2):
  total = ids2.shape[0] * CH
  b_per_w = total // NW
  n_ch = b_per_w // CH
  mesh = plsc.VectorSubcoreMesh(
      core_axis_name="c", subcore_axis_name="s", num_cores=NC, num_subcores=NS
  )

  n_win = n_ch + 8 - n_ch % 8 if n_ch % 8 else n_ch

  @functools.partial(
      pl.kernel,
      out_type=jax.ShapeDtypeStruct((total, D), jnp.float32),
      mesh=mesh,
      scratch_types=[
          pltpu.VMEM((n_win, CH), jnp.int32),
          pltpu.VMEM((CH, D), jnp.float32),
          pltpu.VMEM((CH, D), jnp.float32),
          pltpu.SemaphoreType.DMA,
          pltpu.SemaphoreType.DMA,
      ],
  )
  def k(table_hbm, ids_hbm, out_hbm, idx_v, rows0, rows1, sem0, sem1):
    wid = lax.axis_index("s") * NC + lax.axis_index("c")
    base = wid * b_per_w
    row0 = wid * n_ch
    win0 = (row0 // 8) * 8
    s0 = row0 - win0
    pltpu.sync_copy(ids_hbm.at[pl.ds(win0, n_win)], idx_v)
    pltpu.async_copy(table_hbm.at[idx_v.at[s0]], rows0, sem0)
    pltpu.async_copy(table_hbm.at[idx_v.at[s0 + 1]], rows1, sem1)

    def pair(p, _):
      g0 = 2 * p

      pltpu.make_async_copy(table_hbm.at[idx_v.at[s0 + g0]], rows0, sem0).wait()
      pltpu.sync_copy(rows0, out_hbm.at[pl.ds(base + g0 * CH, CH)])

      @pl.when(g0 + 2 < n_ch)
      def _():
        pltpu.async_copy(table_hbm.at[idx_v.at[s0 + g0 + 2]], rows0, sem0)

      pltpu.make_async_copy(
          table_hbm.at[idx_v.at[s0 + g0 + 1]], rows1, sem1).wait()
      pltpu.sync_copy(rows1, out_hbm.at[pl.ds(base + (g0 + 1) * CH, CH)])

      @pl.when(g0 + 3 < n_ch)
      def _():
        pltpu.async_copy(table_hbm.at[idx_v.at[s0 + g0 + 3]], rows1, sem1)

      return ()

    lax.fori_loop(0, n_ch // 2, pair, ())

  return k(table, ids2)


def _tc_main(x, g, wa, ba, wt, bt, rows_per_tile):
  bl = x.shape[0]
  nt = bl // rows_per_tile
  r = rows_per_tile

  def body(x_ref, g_ref, wa_ref, ba_ref, wt_ref, bt_ref, out_ref, m_s):
    @pl.when(pl.program_id(0) == 0)
    def _():
      m_s[...] = lax.dot_general(
          wa_ref[...], wt_ref[...], (((0,), (0,)), ((), ())),
          preferred_element_type=jnp.float32)

    xv = x_ref[...]
    gv = g_ref[...]
    v = jnp.dot(ba_ref[...], wt_ref[...], preferred_element_type=jnp.float32)
    u = jnp.dot(bt_ref[...], wa_ref[...], preferred_element_type=jnp.float32)
    c = jnp.sum(ba_ref[...] * bt_ref[...])
    a = jnp.dot(xv, m_s[...], preferred_element_type=jnp.float32) + v
    res = jnp.sum(a * gv, axis=1) + jnp.sum(xv * u, axis=1) + c
    out_ref[...] = res.reshape(1, 1, r)

  out = pl.pallas_call(
      body,
      grid=(nt,),
      in_specs=[
          pl.BlockSpec((r, D), lambda i: (i, 0)),
          pl.BlockSpec((r, D), lambda i: (i, 0)),
          pl.BlockSpec((D, D), lambda i: (0, 0)),
          pl.BlockSpec((1, D), lambda i: (0, 0)),
          pl.BlockSpec((D, D), lambda i: (0, 0)),
          pl.BlockSpec((1, D), lambda i: (0, 0)),
      ],
      out_specs=pl.BlockSpec((1, 1, r), lambda i: (i, 0, 0)),
      out_shape=jax.ShapeDtypeStruct((nt, 1, r), jnp.float32),
      scratch_shapes=[pltpu.VMEM((D, D), jnp.float32)],
  )(x, g, wa, ba, wt, bt)
  return out.reshape(bl)


def kernel(actor_emb, topic_ids, Wa, ba, table, Wt, bt, scale):
  b, l, d = actor_emb.shape
  bl = b * l

  wa_s = Wa * scale
  ba_s = (ba * scale).reshape(1, d)

  ids2 = topic_ids.astype(jnp.int32).reshape(bl // CH, CH)
  gathered = _sc_gather(table, ids2)
  x = actor_emb.reshape(bl, d)
  out = _tc_main(x, gathered, wa_s, ba_s, Wt, bt.reshape(1, d), 2048)
  return out.reshape(b, l)


# speedup vs baseline: 4.3688x; 1.9194x over previous
import functools

import jax
import jax.numpy as jnp
from jax import lax
from jax.experimental import pallas as pl
from jax.experimental.pallas import tpu as pltpu
from jax.experimental.pallas import tpu_sc as plsc

D = 128
NC = 2
NS = 16
NW = NC * NS
CH = 128


def _sc_gather(table, ids_t):
  n_ch, b = ids_t.shape
  total = n_ch * b
  mesh = plsc.VectorSubcoreMesh(
      core_axis_name="c", subcore_axis_name="s", num_cores=NC, num_subcores=NS
  )

  @functools.partial(
      pl.kernel,
      out_type=jax.ShapeDtypeStruct((total, D), jnp.float32),
      mesh=mesh,
      scratch_types=[
          pltpu.VMEM((n_ch, CH), jnp.int32),
          pltpu.VMEM((CH, D), jnp.float32),
          pltpu.VMEM((CH, D), jnp.float32),
          pltpu.SemaphoreType.DMA,
          pltpu.SemaphoreType.DMA,
      ],
  )
  def k(table_hbm, ids_hbm, out_hbm, idx_v, rows0, rows1, sem0, sem1):
    wid = lax.axis_index("s") * NC + lax.axis_index("c")
    col0 = wid * CH
    pltpu.sync_copy(ids_hbm.at[pl.ds(0, n_ch), pl.ds(col0, CH)], idx_v)
    pltpu.async_copy(table_hbm.at[idx_v.at[0]], rows0, sem0)
    pltpu.async_copy(table_hbm.at[idx_v.at[1]], rows1, sem1)

    def pair(p, _):
      g0 = 2 * p

      pltpu.make_async_copy(table_hbm.at[idx_v.at[g0]], rows0, sem0).wait()
      pltpu.sync_copy(rows0, out_hbm.at[pl.ds(g0 * b + col0, CH)])

      @pl.when(g0 + 2 < n_ch)
      def _():
        pltpu.async_copy(table_hbm.at[idx_v.at[g0 + 2]], rows0, sem0)

      pltpu.make_async_copy(
          table_hbm.at[idx_v.at[g0 + 1]], rows1, sem1).wait()
      pltpu.sync_copy(rows1, out_hbm.at[pl.ds((g0 + 1) * b + col0, CH)])

      @pl.when(g0 + 3 < n_ch)
      def _():
        pltpu.async_copy(table_hbm.at[idx_v.at[g0 + 3]], rows1, sem1)

      return ()

    lax.fori_loop(0, n_ch // 2, pair, ())

  return k(table, ids_t)


def _tc_main(x, g, wa, ba, wt, bt, rows_per_tile):
  bl = x.shape[0]
  nt = bl // rows_per_tile
  r = rows_per_tile

  def body(x_ref, g_ref, wa_ref, ba_ref, wt_ref, bt_ref, out_ref, m_s):
    @pl.when(pl.program_id(0) == 0)
    def _():
      m_s[...] = lax.dot_general(
          wa_ref[...], wt_ref[...], (((0,), (0,)), ((), ())),
          preferred_element_type=jnp.float32)

    xv = x_ref[...]
    gv = g_ref[...]
    v = jnp.dot(ba_ref[...], wt_ref[...], preferred_element_type=jnp.float32)
    u = jnp.dot(bt_ref[...], wa_ref[...], preferred_element_type=jnp.float32)
    c = jnp.sum(ba_ref[...] * bt_ref[...])
    a = jnp.dot(xv, m_s[...], preferred_element_type=jnp.float32) + v
    ones = jnp.ones((1, D), dtype=jnp.float32)
    res = lax.dot_general(
        ones, a * gv, (((1,), (1,)), ((), ())),
        preferred_element_type=jnp.float32)
    z = lax.dot_general(
        u, xv, (((1,), (1,)), ((), ())),
        preferred_element_type=jnp.float32)
    out_ref[...] = (res + z + c).reshape(1, 1, r)

  out = pl.pallas_call(
      body,
      grid=(nt,),
      in_specs=[
          pl.BlockSpec((r, D), lambda i: (i, 0)),
          pl.BlockSpec((r, D), lambda i: (i, 0)),
          pl.BlockSpec((D, D), lambda i: (0, 0)),
          pl.BlockSpec((1, D), lambda i: (0, 0)),
          pl.BlockSpec((D, D), lambda i: (0, 0)),
          pl.BlockSpec((1, D), lambda i: (0, 0)),
      ],
      out_specs=pl.BlockSpec((1, 1, r), lambda i: (i, 0, 0)),
      out_shape=jax.ShapeDtypeStruct((nt, 1, r), jnp.float32),
      scratch_shapes=[pltpu.VMEM((D, D), jnp.float32)],
  )(x, g, wa, ba, wt, bt)
  return out.reshape(bl)


def kernel(actor_emb, topic_ids, Wa, ba, table, Wt, bt, scale):
  b, l, d = actor_emb.shape
  bl = b * l

  wa_s = Wa * scale
  ba_s = (ba * scale).reshape(1, d)

  ids_t = topic_ids.T.astype(jnp.int32)
  x = actor_emb.transpose(1, 0, 2).reshape(bl, d)
  gathered = _sc_gather(table, ids_t)
  out = _tc_main(x, gathered, wa_s, ba_s, Wt, bt.reshape(1, d), 2048)
  return out.reshape(l, b).T
